# Initial kernel scaffold; baseline (speedup 1.0000x reference)
import jax
import jax.numpy as jnp
from jax.experimental import pallas as pl


def kernel(h, x, edge_index, edge_kernel, Wz, bz, Wr, br, Wq, bq):
    raise NotImplementedError("write your pallas kernel here")


# speedup vs baseline: 3.3702x; 3.3702x over previous
import functools

import jax
import jax.numpy as jnp
from jax import lax
from jax.experimental import pallas as pl
from jax.experimental.pallas import tpu as pltpu
from jax.experimental.pallas import tpu_sc as plsc

_N = 10000
_K = 27
_HID = 128
_CIN = 256
_E = 160000
_NK = _N * _K
_KH = _K * _HID

_BLK = 128
_NBLK = _E // _BLK
_NSUB = 16
_BN = 400
_BD = 1152

_mesh = plsc.VectorSubcoreMesh(core_axis_name="c", subcore_axis_name="s")


def _make_sc_pass(table_rows: int, add_core_offset: bool, split_blocks: bool):
    def body(src_hbm, kern_hbm, dst_hbm, tab_hbm, out_hbm,
             src_v, kern_v, dst_v, idx_v, rows_v, accum, sem):
        c = lax.axis_index("c")
        s = lax.axis_index("s")

        def _zrow(i, _):
            r = i // 8
            col = (i % 8) * 16
            rows_v[r, pl.ds(col, 16)] = jnp.zeros((16,), jnp.float32)
            return 0
        lax.fori_loop(0, _BLK * 8, _zrow, 0)

        @pl.when(s < 10)
        def _init():
            base = s * 1000
            for j in range(7):
                pltpu.sync_copy(rows_v, accum.at[pl.ds(base + j * 128, 128)])
            pltpu.sync_copy(rows_v.at[pl.ds(0, 104)],
                            accum.at[pl.ds(base + 896, 104)])

        plsc.subcore_barrier()

        if split_blocks:
            nb_total = _NBLK // 2
            base_blk = c * nb_total
        else:
            nb_total = _NBLK
            base_blk = s * 0
        nb = (nb_total - 1 - s) // _NSUB + 1

        def step(i, _):
            b = base_blk + s + i * _NSUB
            off = b * _BLK
            pltpu.sync_copy(src_hbm.at[pl.ds(off, _BLK)], src_v)
            pltpu.sync_copy(kern_hbm.at[pl.ds(off, _BLK)], kern_v)
            pltpu.sync_copy(dst_hbm.at[pl.ds(off, _BLK)], dst_v)
            for j in range(_BLK // 16):
                sl = pl.ds(j * 16, 16)
                gi = src_v[sl] * _K + kern_v[sl]
                if add_core_offset:
                    gi = gi + c * _NK
                idx_v[sl] = gi
            pltpu.async_copy(tab_hbm.at[idx_v], rows_v, sem).wait()
            pltpu.sync_copy(rows_v, accum.at[dst_v], add=True)
            return 0
        lax.fori_loop(0, nb, step, 0)

        plsc.subcore_barrier()

        @pl.when(s < 10)
        def _flush():
            base = s * 1000
            pltpu.sync_copy(accum.at[pl.ds(base, 1000)],
                            out_hbm.at[pl.ds(c * _N + base, 1000)])

    return pl.kernel(
        body,
        out_type=jax.ShapeDtypeStruct((2 * _N, _HID), jnp.float32),
        mesh=_mesh,
        scratch_types=[
            pltpu.VMEM((_BLK,), jnp.int32),
            pltpu.VMEM((_BLK,), jnp.int32),
            pltpu.VMEM((_BLK,), jnp.int32),
            pltpu.VMEM((_BLK,), jnp.int32),
            pltpu.VMEM((_BLK, _HID), jnp.float32),
            pltpu.VMEM_SHARED((_N, _HID), jnp.float32),
            pltpu.SemaphoreType.DMA,
        ],
    )


_sc_pass_zr = _make_sc_pass(2 * _NK, add_core_offset=True, split_blocks=False)
_sc_pass_q = _make_sc_pass(_NK, add_core_offset=False, split_blocks=True)


def _zr_mm_body(hx_ref, w_ref, out_ref):
    out_ref[0] = jnp.dot(hx_ref[...], w_ref[0],
                         preferred_element_type=jnp.float32)


_zr_mm = pl.pallas_call(
    _zr_mm_body,
    grid=(2, _KH // _BD, _N // _BN),
    in_specs=[
        pl.BlockSpec((_BN, _CIN), lambda zr, j, i: (i, 0)),
        pl.BlockSpec((1, _CIN, _BD), lambda zr, j, i: (zr, 0, j)),
    ],
    out_specs=pl.BlockSpec((1, _BN, _BD), lambda zr, j, i: (zr, i, j)),
    out_shape=jax.ShapeDtypeStruct((2, _N, _KH), jnp.float32),
)


def _q_mm_body(pr_ref, h_ref, x_ref, br_ref, wh_ref, wx_ref, out_ref):
    r = jax.nn.sigmoid(pr_ref[...] + br_ref[0])
    rh = r * h_ref[...]
    out_ref[...] = (
        jnp.dot(rh, wh_ref[...], preferred_element_type=jnp.float32)
        + jnp.dot(x_ref[...], wx_ref[...], preferred_element_type=jnp.float32))


_q_mm = pl.pallas_call(
    _q_mm_body,
    grid=(_KH // _BD, _N // _BN),
    in_specs=[
        pl.BlockSpec((_BN, _HID), lambda j, i: (i, 0)),
        pl.BlockSpec((_BN, _HID), lambda j, i: (i, 0)),
        pl.BlockSpec((_BN, _HID), lambda j, i: (i, 0)),
        pl.BlockSpec((1, _HID), lambda j, i: (0, 0)),
        pl.BlockSpec((_HID, _BD), lambda j, i: (0, j)),
        pl.BlockSpec((_HID, _BD), lambda j, i: (0, j)),
    ],
    out_specs=pl.BlockSpec((_BN, _BD), lambda j, i: (i, j)),
    out_shape=jax.ShapeDtypeStruct((_N, _KH), jnp.float32),
)


def _gate_body(pz_ref, q0_ref, q1_ref, h_ref, bz_ref, bq_ref, out_ref):
    z = jax.nn.sigmoid(pz_ref[...] + bz_ref[0])
    q = jnp.tanh(q0_ref[...] + q1_ref[...] + bq_ref[0])
    out_ref[...] = (1.0 - z) * h_ref[...] + z * q


_gate = pl.pallas_call(
    _gate_body,
    grid=(_N // _BN,),
    in_specs=[
        pl.BlockSpec((_BN, _HID), lambda i: (i, 0)),
        pl.BlockSpec((_BN, _HID), lambda i: (i, 0)),
        pl.BlockSpec((_BN, _HID), lambda i: (i, 0)),
        pl.BlockSpec((_BN, _HID), lambda i: (i, 0)),
        pl.BlockSpec((1, _HID), lambda i: (0, 0)),
        pl.BlockSpec((1, _HID), lambda i: (0, 0)),
    ],
    out_specs=pl.BlockSpec((_BN, _HID), lambda i: (i, 0)),
    out_shape=jax.ShapeDtypeStruct((_N, _HID), jnp.float32),
)


def kernel(h, x, edge_index, edge_kernel, Wz, bz, Wr, br, Wq, bq):
    hx = jnp.concatenate([h, x], axis=1)
    wzf = Wz.transpose(1, 0, 2).reshape(_CIN, _KH)
    wrf = Wr.transpose(1, 0, 2).reshape(_CIN, _KH)
    wzr = jnp.stack([wzf, wrf])
    wqh = Wq[:, :_HID, :].transpose(1, 0, 2).reshape(_HID, _KH)
    wqx = Wq[:, _HID:, :].transpose(1, 0, 2).reshape(_HID, _KH)

    src = edge_index[0]
    dst = edge_index[1]

    t1 = _zr_mm(hx, wzr).reshape(2 * _NK, _HID)
    pre = _sc_pass_zr(src, edge_kernel, dst, t1)
    pz, pr = pre[:_N], pre[_N:]
    t2 = _q_mm(pr, h, x, br.reshape(1, _HID), wqh, wqx).reshape(_NK, _HID)
    qp = _sc_pass_q(src, edge_kernel, dst, t2)
    return _gate(pz, qp[:_N], qp[_N:], h,
                 bz.reshape(1, _HID), bq.reshape(1, _HID))
